# baseline (speedup 1.0000x reference)
import functools

import jax
import jax.numpy as jnp
from jax import lax
from jax.experimental import pallas as pl
from jax.experimental.pallas import tpu as pltpu
from jax.experimental.pallas import tpu_sc as plsc

NC = 2
NS = 16
NW = NC * NS
CHUNK = 128
NBUF = 10
LEAD = 5


def kernel(input_, weight):
    B, S = input_.shape
    _, D = weight.shape
    total = B * S
    assert total % (NW * CHUNK) == 0
    n_chunks = total // (NW * CHUNK)
    assert n_chunks % NBUF == 0
    n_rounds = n_chunks // NBUF

    idx = input_.reshape(NW, n_chunks, CHUNK).astype(jnp.int32)

    mesh = plsc.VectorSubcoreMesh(
        core_axis_name="c", subcore_axis_name="s", num_cores=NC, num_subcores=NS
    )

    @functools.partial(
        pl.kernel,
        out_type=jax.ShapeDtypeStruct((NW, n_chunks, CHUNK, D), jnp.float32),
        mesh=mesh,
        scratch_types=[
            pltpu.VMEM((n_chunks, CHUNK), jnp.int32),
            pltpu.VMEM((NBUF, CHUNK, D), jnp.float32),
            pltpu.SemaphoreType.DMA((NBUF,)),
            pltpu.SemaphoreType.DMA((NBUF,)),
        ],
        compiler_params=pltpu.CompilerParams(use_tc_tiling_on_sc=False),
    )
    def emb(idx_hbm, w_hbm, out_hbm, idx_v, rows_v, gsem, wsem):
        wid = lax.axis_index("s") * NC + lax.axis_index("c")
        pltpu.sync_copy(idx_hbm.at[wid], idx_v)

        for b in range(LEAD):
            pltpu.async_copy(w_hbm.at[idx_v.at[b]], rows_v.at[b], gsem.at[b])

        @pl.loop(0, n_rounds)
        def body(g):
            for b in range(NBUF):
                cur = g * NBUF + b
                pb = (b + LEAD) % NBUF
                pltpu.make_async_copy(
                    w_hbm.at[idx_v.at[cur]], rows_v.at[b], gsem.at[b]
                ).wait()
                pltpu.async_copy(rows_v.at[b], out_hbm.at[wid, cur], wsem.at[b])

                def issue(cur=cur, b=b, pb=pb, drain=True):
                    if drain:
                        pltpu.make_async_copy(
                            rows_v.at[pb], out_hbm.at[wid, cur - LEAD], wsem.at[pb]
                        ).wait()
                    pltpu.async_copy(
                        w_hbm.at[idx_v.at[cur + LEAD]], rows_v.at[pb], gsem.at[pb]
                    )

                if b < LEAD:
                    @pl.when(g > 0)
                    def _():
                        issue(drain=True)

                    @pl.when(g == 0)
                    def _():
                        issue(drain=False)
                else:
                    @pl.when(g < n_rounds - 1)
                    def _():
                        issue(drain=True)

        for b in range(NBUF):
            pltpu.make_async_copy(
                rows_v.at[b], out_hbm.at[wid, n_chunks - NBUF + b], wsem.at[b]
            ).wait()

    out = emb(idx, weight)
    return out.reshape(B, S, D)
